# baseline (speedup 1.0000x reference)
import functools

import jax
import jax.numpy as jnp
from jax import lax
from jax.experimental import pallas as pl
from jax.experimental.pallas import tpu as pltpu
from jax.experimental.pallas import tpu_sc as plsc

NUM_EMB = 10
EMB_DIM = 4
NUM_CLS = 5
LANES = 16
TAB_PAD = 64


def kernel(x, y, emb_table, fc_w, fc_b):
    B, L = x.shape
    N = B * L

    info = plsc.get_sparse_core_info()
    NC, NS = info.num_cores, info.num_subcores
    NW = NC * NS
    assert N % NW == 0
    per_w = N // NW
    CHUNK = 12800
    assert per_w % CHUNK == 0
    n_chunks = per_w // CHUNK
    NBUF = 2
    UNROLL = 8
    NACC = 4

    def table_body(emb_ref, fcw_ref, fcb_ref, w_ref):
        logits = lax.dot_general(
            emb_ref[...], fcw_ref[...], (((1,), (1,)), ((), ())),
            preferred_element_type=jnp.float32,
        )
        logits = logits + fcb_ref[...]
        m = jnp.max(logits, axis=1, keepdims=True)
        ex = jnp.exp(logits - m)
        prop = ex / jnp.sum(ex, axis=1, keepdims=True)
        m2 = jnp.max(prop, axis=1, keepdims=True)
        lse = jnp.log(jnp.sum(jnp.exp(prop - m2), axis=1, keepdims=True)) + m2
        w_ref[...] = lse - prop

    w_tab = pl.pallas_call(
        table_body,
        out_shape=jax.ShapeDtypeStruct((NUM_EMB, NUM_CLS), jnp.float32),
    )(emb_table, fc_w, fc_b.reshape(1, NUM_CLS))

    w_flat = jnp.pad(w_tab.reshape(-1), (0, TAB_PAD - NUM_EMB * NUM_CLS))

    mesh = plsc.VectorSubcoreMesh(core_axis_name="c", subcore_axis_name="s")

    CROWS = CHUNK // L
    assert CROWS * L == CHUNK

    def make_lookup(rows_call, tok_off):
        rpw = rows_call // NW
        assert rpw % CROWS == 0
        nck = rpw // CROWS
        pw = rpw * L

        def lookup_sum(x_hbm, y_hbm, w_hbm, out_hbm,
                       wbuf, xbuf0, xbuf1, ybuf0, ybuf1, accbuf,
                       xsems, ysems):
            wid = lax.axis_index("s") * NC + lax.axis_index("c")
            pltpu.sync_copy(w_hbm, wbuf)
            xbufs, ybufs = [xbuf0, xbuf1], [ybuf0, ybuf1]
            row0 = wid * rpw
            tok0 = tok_off + wid * pw

            def start(ci):
                b = ci % NBUF
                return (
                    pltpu.async_copy(
                        x_hbm.at[pl.ds(row0 + ci * CROWS, CROWS)],
                        xbufs[b], xsems.at[b]),
                    pltpu.async_copy(
                        y_hbm.at[pl.ds(tok0 + ci * CHUNK, CHUNK)],
                        ybufs[b], ysems.at[b]),
                )

            lane = lax.iota(jnp.int32, LANES)
            str_row = (lane >= 8).astype(jnp.int32)
            str_col = jnp.where(lane < 8, lane + (L - 8), lane - 8)

            VPB = 2 * L // LANES
            n_steps = CHUNK // (2 * L)

            inflight = {0: start(0)}
            accs = [jnp.zeros((LANES,), jnp.float32) for _ in range(NACC)]
            for ci in range(nck):
                if ci + 1 < nck:
                    inflight[ci + 1] = start(ci + 1)
                for h in inflight.pop(ci):
                    h.wait()
                b = ci % NBUF
                xb, yb = xbufs[b], ybufs[b]

                def body(i, a):
                    a = list(a)
                    rbase = 2 * i
                    for u in range(VPB):
                        t0 = u * LANES
                        col0, roff = t0 % L, t0 // L
                        if col0 + LANES <= L:
                            row_vec = jnp.full((LANES,), rbase + roff,
                                               jnp.int32)
                            col_vec = lane + col0
                        else:
                            row_vec = str_row + (rbase + roff)
                            col_vec = str_col
                        xv = plsc.load_gather(xb, [row_vec, col_vec])
                        yv = yb[pl.ds(i * (2 * L) + t0, LANES)]
                        k = xv * NUM_CLS + yv
                        a[u % NACC] = a[u % NACC] + plsc.load_gather(wbuf, [k])
                    return tuple(a)

                accs = lax.fori_loop(0, n_steps, body, tuple(accs))
            acc = accs[0]
            for a in accs[1:]:
                acc = acc + a
            accbuf[...] = acc
            pltpu.sync_copy(accbuf, out_hbm.at[wid])

        return functools.partial(
            pl.kernel,
            mesh=mesh,
            out_type=jax.ShapeDtypeStruct((NW, LANES), jnp.float32),
            scratch_types=[
                pltpu.VMEM((TAB_PAD,), jnp.float32),
                pltpu.VMEM((CROWS, L), jnp.int32),
                pltpu.VMEM((CROWS, L), jnp.int32),
                pltpu.VMEM((CHUNK,), jnp.int32),
                pltpu.VMEM((CHUNK,), jnp.int32),
                pltpu.VMEM((LANES,), jnp.float32),
                pltpu.SemaphoreType.DMA((NBUF,)),
                pltpu.SemaphoreType.DMA((NBUF,)),
            ],
            compiler_params=pltpu.CompilerParams(needs_layout_passes=False),
        )(lookup_sum)

    ROWS_A = 4096
    pA = make_lookup(ROWS_A, 0)(x[:ROWS_A], y, w_flat)
    pB = make_lookup(B - ROWS_A, ROWS_A * L)(x[ROWS_A:], y, w_flat)

    def combine_body(pa_ref, pb_ref, out_ref):
        s = jnp.sum(pa_ref[...]) + jnp.sum(pb_ref[...])
        out_ref[...] = jnp.full((1, 1), s / jnp.float32(N), jnp.float32)

    out = pl.pallas_call(
        combine_body,
        out_shape=jax.ShapeDtypeStruct((1, 1), jnp.float32),
    )(pA, pB)
    return out[0, 0]


# speedup vs baseline: 1.2321x; 1.2321x over previous
import functools

import jax
import jax.numpy as jnp
from jax import lax
from jax.experimental import pallas as pl
from jax.experimental.pallas import tpu as pltpu
from jax.experimental.pallas import tpu_sc as plsc

NUM_EMB = 10
EMB_DIM = 4
NUM_CLS = 5
LANES = 16
TAB_PAD = 64


def kernel(x, y, emb_table, fc_w, fc_b):
    B, L = x.shape
    N = B * L

    info = plsc.get_sparse_core_info()
    NC, NS = info.num_cores, info.num_subcores
    NW = NC * NS
    assert N % NW == 0
    per_w = N // NW
    CHUNK = 25600
    assert per_w % CHUNK == 0
    n_chunks = per_w // CHUNK
    NBUF = 2
    UNROLL = 8
    NACC = 4

    def table_body(emb_ref, fcw_ref, fcb_ref, w_ref):
        logits = lax.dot_general(
            emb_ref[...], fcw_ref[...], (((1,), (1,)), ((), ())),
            preferred_element_type=jnp.float32,
        )
        logits = logits + fcb_ref[...]
        m = jnp.max(logits, axis=1, keepdims=True)
        ex = jnp.exp(logits - m)
        prop = ex / jnp.sum(ex, axis=1, keepdims=True)
        m2 = jnp.max(prop, axis=1, keepdims=True)
        lse = jnp.log(jnp.sum(jnp.exp(prop - m2), axis=1, keepdims=True)) + m2
        w_ref[...] = lse - prop

    w_tab = pl.pallas_call(
        table_body,
        out_shape=jax.ShapeDtypeStruct((NUM_EMB, NUM_CLS), jnp.float32),
    )(emb_table, fc_w, fc_b.reshape(1, NUM_CLS))

    w_flat = jnp.pad(w_tab.reshape(-1), (0, TAB_PAD - NUM_EMB * NUM_CLS))

    mesh = plsc.VectorSubcoreMesh(core_axis_name="c", subcore_axis_name="s")

    rows_per_w = B // NW
    CROWS = CHUNK // L
    assert CROWS * L == CHUNK and rows_per_w % CROWS == 0

    def lookup_sum(x_hbm, y_hbm, w_hbm, out_hbm,
                   wbuf, xbuf0, xbuf1, ybuf0, ybuf1, accbuf, xsems, ysems):
        wid = lax.axis_index("s") * NC + lax.axis_index("c")
        pltpu.sync_copy(w_hbm, wbuf)
        xbufs, ybufs = [xbuf0, xbuf1], [ybuf0, ybuf1]
        row0 = wid * rows_per_w
        tok0 = wid * per_w

        def start(ci):
            b = ci % NBUF
            return (
                pltpu.async_copy(
                    x_hbm.at[pl.ds(row0 + ci * CROWS, CROWS)],
                    xbufs[b], xsems.at[b]),
                pltpu.async_copy(
                    y_hbm.at[pl.ds(tok0 + ci * CHUNK, CHUNK)],
                    ybufs[b], ysems.at[b]),
            )

        lane = lax.iota(jnp.int32, LANES)
        str_row = (lane >= 8).astype(jnp.int32)
        str_col = jnp.where(lane < 8, lane + (L - 8), lane - 8)

        VPB = 2 * L // LANES
        n_steps = CHUNK // (2 * L)

        inflight = {0: start(0)}
        accs = [jnp.zeros((LANES,), jnp.float32) for _ in range(NACC)]
        for ci in range(n_chunks):
            if ci + 1 < n_chunks:
                inflight[ci + 1] = start(ci + 1)
            for h in inflight.pop(ci):
                h.wait()
            b = ci % NBUF
            xb, yb = xbufs[b], ybufs[b]

            def body(i, a):
                a = list(a)
                rbase = 2 * i
                for u in range(VPB):
                    t0 = u * LANES
                    col0, roff = t0 % L, t0 // L
                    if col0 + LANES <= L:
                        row_vec = jnp.full((LANES,), rbase + roff, jnp.int32)
                        col_vec = lane + col0
                    else:
                        row_vec = str_row + (rbase + roff)
                        col_vec = str_col
                    xv = plsc.load_gather(xb, [row_vec, col_vec])
                    yv = yb[pl.ds(i * (2 * L) + t0, LANES)]
                    k = xv * NUM_CLS + yv
                    a[u % NACC] = a[u % NACC] + plsc.load_gather(wbuf, [k])
                return tuple(a)

            accs = lax.fori_loop(0, n_steps, body, tuple(accs))
        acc = accs[0]
        for a in accs[1:]:
            acc = acc + a
        accbuf[...] = acc
        pltpu.sync_copy(accbuf, out_hbm.at[wid])

    lookup_sum = functools.partial(
        pl.kernel,
        mesh=mesh,
        out_type=jax.ShapeDtypeStruct((NW, LANES), jnp.float32),
        scratch_types=[
            pltpu.VMEM((TAB_PAD,), jnp.float32),
            pltpu.VMEM((CROWS, L), jnp.int32),
            pltpu.VMEM((CROWS, L), jnp.int32),
            pltpu.VMEM((CHUNK,), jnp.int32),
            pltpu.VMEM((CHUNK,), jnp.int32),
            pltpu.VMEM((LANES,), jnp.float32),
            pltpu.SemaphoreType.DMA((NBUF,)),
            pltpu.SemaphoreType.DMA((NBUF,)),
        ],
        compiler_params=pltpu.CompilerParams(needs_layout_passes=False),
    )(lookup_sum)

    partials = lookup_sum(x, y, w_flat)

    def combine_body(p_ref, out_ref):
        out_ref[...] = jnp.full(
            (1, 1), jnp.sum(p_ref[...]) / jnp.float32(N), jnp.float32)

    out = pl.pallas_call(
        combine_body,
        out_shape=jax.ShapeDtypeStruct((1, 1), jnp.float32),
    )(partials)
    return out[0, 0]


# speedup vs baseline: 1.2698x; 1.0306x over previous
import functools

import jax
import jax.numpy as jnp
from jax import lax
from jax.experimental import pallas as pl
from jax.experimental.pallas import tpu as pltpu
from jax.experimental.pallas import tpu_sc as plsc

NUM_EMB = 10
EMB_DIM = 4
NUM_CLS = 5
LANES = 16
TAB_PAD = 64


def kernel(x, y, emb_table, fc_w, fc_b):
    B, L = x.shape
    N = B * L

    info = plsc.get_sparse_core_info()
    NC, NS = info.num_cores, info.num_subcores
    NW = NC * NS
    assert N % NW == 0
    per_w = N // NW
    CHUNK = 12800
    assert per_w % CHUNK == 0
    n_chunks = per_w // CHUNK
    NBUF = 2
    UNROLL = 8
    NACC = 4

    def table_body(emb_ref, fcw_ref, fcb_ref, w_ref):
        logits = lax.dot_general(
            emb_ref[...], fcw_ref[...], (((1,), (1,)), ((), ())),
            preferred_element_type=jnp.float32,
        )
        logits = logits + fcb_ref[...].reshape(1, NUM_CLS)
        m = jnp.max(logits, axis=1, keepdims=True)
        ex = jnp.exp(logits - m)
        prop = ex / jnp.sum(ex, axis=1, keepdims=True)
        m2 = jnp.max(prop, axis=1, keepdims=True)
        lse = jnp.log(jnp.sum(jnp.exp(prop - m2), axis=1, keepdims=True)) + m2
        w_ref[...] = lse - prop

    w_tab = pl.pallas_call(
        table_body,
        out_shape=jax.ShapeDtypeStruct((NUM_EMB, NUM_CLS), jnp.float32),
    )(emb_table, fc_w, fc_b)

    w_flat = jnp.pad(w_tab.reshape(-1), (0, TAB_PAD - NUM_EMB * NUM_CLS))

    mesh = plsc.VectorSubcoreMesh(core_axis_name="c", subcore_axis_name="s")

    rows_per_w = B // NW
    CROWS = CHUNK // L
    assert CROWS * L == CHUNK and rows_per_w % CROWS == 0

    def lookup_sum(x_hbm, y_hbm, w_hbm, out_hbm,
                   wbuf, xbuf0, xbuf1, ybuf0, ybuf1, accbuf, xsems, ysems):
        wid = lax.axis_index("s") * NC + lax.axis_index("c")
        pltpu.sync_copy(w_hbm, wbuf)
        xbufs, ybufs = [xbuf0, xbuf1], [ybuf0, ybuf1]
        row0 = wid * rows_per_w
        tok0 = wid * per_w

        def start(ci):
            b = ci % NBUF
            return (
                pltpu.async_copy(
                    x_hbm.at[pl.ds(row0 + ci * CROWS, CROWS)],
                    xbufs[b], xsems.at[b]),
                pltpu.async_copy(
                    y_hbm.at[pl.ds(tok0 + ci * CHUNK, CHUNK)],
                    ybufs[b], ysems.at[b]),
            )

        lane = lax.iota(jnp.int32, LANES)
        str_row = (lane >= 8).astype(jnp.int32)
        str_col = jnp.where(lane < 8, lane + (L - 8), lane - 8)

        VPB = 2 * L // LANES
        n_steps = CHUNK // (2 * L)

        inflight = {0: start(0)}
        accs = [jnp.zeros((LANES,), jnp.float32) for _ in range(NACC)]
        for ci in range(n_chunks):
            if ci + 1 < n_chunks:
                inflight[ci + 1] = start(ci + 1)
            for h in inflight.pop(ci):
                h.wait()
            b = ci % NBUF
            xb, yb = xbufs[b], ybufs[b]

            def body(i, a):
                a = list(a)
                rbase = 2 * i
                for u in range(VPB):
                    t0 = u * LANES
                    col0, roff = t0 % L, t0 // L
                    if col0 + LANES <= L:
                        row_vec = jnp.full((LANES,), rbase + roff, jnp.int32)
                        col_vec = lane + col0
                    else:
                        row_vec = str_row + (rbase + roff)
                        col_vec = str_col
                    xv = plsc.load_gather(xb, [row_vec, col_vec])
                    yv = yb[pl.ds(i * (2 * L) + t0, LANES)]
                    k = xv * NUM_CLS + yv
                    a[u % NACC] = a[u % NACC] + plsc.load_gather(wbuf, [k])
                return tuple(a)

            accs = lax.fori_loop(0, n_steps, body, tuple(accs))
        acc = accs[0]
        for a in accs[1:]:
            acc = acc + a
        accbuf[...] = acc
        pltpu.sync_copy(accbuf, out_hbm.at[wid])

    lookup_sum = functools.partial(
        pl.kernel,
        mesh=mesh,
        out_type=jax.ShapeDtypeStruct((NW, LANES), jnp.float32),
        scratch_types=[
            pltpu.VMEM((TAB_PAD,), jnp.float32),
            pltpu.VMEM((CROWS, L), jnp.int32),
            pltpu.VMEM((CROWS, L), jnp.int32),
            pltpu.VMEM((CHUNK,), jnp.int32),
            pltpu.VMEM((CHUNK,), jnp.int32),
            pltpu.VMEM((LANES,), jnp.float32),
            pltpu.SemaphoreType.DMA((NBUF,)),
            pltpu.SemaphoreType.DMA((NBUF,)),
        ],
        compiler_params=pltpu.CompilerParams(needs_layout_passes=False),
    )(lookup_sum)

    partials = lookup_sum(x, y, w_flat)

    def combine_body(p_ref, out_ref):
        out_ref[...] = jnp.full(
            (1, 1), jnp.sum(p_ref[...]) / jnp.float32(N), jnp.float32)

    out = pl.pallas_call(
        combine_body,
        out_shape=jax.ShapeDtypeStruct((1, 1), jnp.float32),
    )(partials)
    return out[0, 0]


# speedup vs baseline: 1.2712x; 1.0011x over previous
import functools

import jax
import jax.numpy as jnp
from jax import lax
from jax.experimental import pallas as pl
from jax.experimental.pallas import tpu as pltpu
from jax.experimental.pallas import tpu_sc as plsc

NUM_EMB = 10
EMB_DIM = 4
NUM_CLS = 5
LANES = 16
TAB_PAD = 64


def kernel(x, y, emb_table, fc_w, fc_b):
    B, L = x.shape
    N = B * L

    info = plsc.get_sparse_core_info()
    NC, NS = info.num_cores, info.num_subcores
    NW = NC * NS
    assert N % NW == 0
    per_w = N // NW
    CHUNK = 12800
    assert per_w % CHUNK == 0
    n_chunks = per_w // CHUNK
    NBUF = 2
    UNROLL = 8
    NACC = 4

    def table_body(emb_ref, fcw_ref, fcb_ref, w_ref):
        logits = lax.dot_general(
            emb_ref[...], fcw_ref[...], (((1,), (1,)), ((), ())),
            preferred_element_type=jnp.float32,
        )
        logits = logits + fcb_ref[...].reshape(1, NUM_CLS)
        m = jnp.max(logits, axis=1, keepdims=True)
        ex = jnp.exp(logits - m)
        prop = ex / jnp.sum(ex, axis=1, keepdims=True)
        m2 = jnp.max(prop, axis=1, keepdims=True)
        lse = jnp.log(jnp.sum(jnp.exp(prop - m2), axis=1, keepdims=True)) + m2
        w_ref[...] = lse - prop

    w_tab = pl.pallas_call(
        table_body,
        out_shape=jax.ShapeDtypeStruct((NUM_EMB, NUM_CLS), jnp.float32),
    )(emb_table, fc_w, fc_b)

    w_flat = jnp.pad(w_tab.reshape(-1), (0, TAB_PAD - NUM_EMB * NUM_CLS))

    mesh = plsc.VectorSubcoreMesh(core_axis_name="c", subcore_axis_name="s")

    rows_per_w = B // NW
    CROWS = CHUNK // L
    assert CROWS * L == CHUNK and rows_per_w % CROWS == 0

    def lookup_sum(x_hbm, y_hbm, w_hbm, out_hbm,
                   wbuf, xbuf0, xbuf1, ybuf0, ybuf1, accbuf, xsems, ysems):
        wid = lax.axis_index("s") * NC + lax.axis_index("c")
        pltpu.sync_copy(w_hbm, wbuf)
        xbufs, ybufs = [xbuf0, xbuf1], [ybuf0, ybuf1]
        row0 = wid * rows_per_w
        tok0 = wid * per_w

        def start(ci):
            b = ci % NBUF
            return (
                pltpu.async_copy(
                    x_hbm.at[pl.ds(row0 + ci * CROWS, CROWS)],
                    xbufs[b], xsems.at[b]),
                pltpu.async_copy(
                    y_hbm.at[pl.ds(tok0 + ci * CHUNK, CHUNK)],
                    ybufs[b], ysems.at[b]),
            )

        lane = lax.iota(jnp.int32, LANES)
        str_row = (lane >= 8).astype(jnp.int32)
        str_col = jnp.where(lane < 8, lane + (L - 8), lane - 8)

        VPB = 2 * L // LANES
        n_steps = CHUNK // (2 * L)

        inflight = {0: start(0)}
        accs = [jnp.zeros((LANES,), jnp.float32) for _ in range(NACC)]
        for ci in range(n_chunks):
            if ci + 1 < n_chunks:
                inflight[ci + 1] = start(ci + 1)
            for h in inflight.pop(ci):
                h.wait()
            b = ci % NBUF
            xb, yb = xbufs[b], ybufs[b]

            def body(i, a):
                a = list(a)
                rbase = 2 * i
                for u in range(VPB):
                    t0 = u * LANES
                    col0, roff = t0 % L, t0 // L
                    if col0 + LANES <= L:
                        row_vec = jnp.full((LANES,), rbase + roff, jnp.int32)
                        col_vec = lane + col0
                    else:
                        row_vec = str_row + (rbase + roff)
                        col_vec = str_col
                    xv = plsc.load_gather(xb, [row_vec, col_vec])
                    yv = yb[pl.ds(i * (2 * L) + t0, LANES)]
                    k = xv * NUM_CLS + yv
                    a[u % NACC] = a[u % NACC] + plsc.load_gather(wbuf, [k])
                return tuple(a)

            accs = lax.fori_loop(0, n_steps, body, tuple(accs))
        acc = accs[0]
        for a in accs[1:]:
            acc = acc + a
        accbuf[...] = acc
        pltpu.sync_copy(accbuf, out_hbm.at[pl.ds(wid * LANES, LANES)])

    lookup_sum = functools.partial(
        pl.kernel,
        mesh=mesh,
        out_type=jax.ShapeDtypeStruct((NW * LANES,), jnp.float32),
        scratch_types=[
            pltpu.VMEM((TAB_PAD,), jnp.float32),
            pltpu.VMEM((CROWS, L), jnp.int32),
            pltpu.VMEM((CROWS, L), jnp.int32),
            pltpu.VMEM((CHUNK,), jnp.int32),
            pltpu.VMEM((CHUNK,), jnp.int32),
            pltpu.VMEM((LANES,), jnp.float32),
            pltpu.SemaphoreType.DMA((NBUF,)),
            pltpu.SemaphoreType.DMA((NBUF,)),
        ],
        compiler_params=pltpu.CompilerParams(needs_layout_passes=False),
    )(lookup_sum)

    partials = lookup_sum(x, y, w_flat)

    def combine_body(p_ref, out_ref):
        out_ref[...] = jnp.full(
            (1, 1), jnp.sum(p_ref[...]) / jnp.float32(N), jnp.float32)

    out = pl.pallas_call(
        combine_body,
        out_shape=jax.ShapeDtypeStruct((1, 1), jnp.float32),
    )(partials)
    return out[0, 0]
